# speedup vs baseline: 2.9360x; 2.9360x over previous
import jax
import jax.numpy as jnp
from jax import lax
from jax.experimental import pallas as pl

_K = 64
_ROWS_PER_BLOCK = 8
_N = 32768


def _lane_cumsum(y, width):
    s = 1
    while s < width:
        shifted = jnp.concatenate(
            [jnp.zeros(y.shape[:-1] + (s,), y.dtype), y[..., :-s]], axis=-1
        )
        y = y + shifted
        s *= 2
    return y


def _topk_body(x_ref, o_ref):
    xb = x_ref[...]
    u = lax.bitcast_convert_type(xb, jnp.int32)
    key = u ^ ((u >> 31) & jnp.int32(0x7FFFFFFF))

    rows = xb.shape[0]

    def step(i, t):
        trial = t + (jnp.int32(1) << (jnp.int32(31) - i))
        cnt = jnp.sum((key >= trial).astype(jnp.int32), axis=1, keepdims=True)
        return jnp.where(cnt >= _K, trial, t)

    t0 = jnp.full((rows, 1), jnp.iinfo(jnp.int32).min, jnp.int32)
    t = lax.fori_loop(0, 32, step, t0)

    gt = key > t
    n_gt = jnp.sum(gt.astype(jnp.int32), axis=1, keepdims=True)
    r = _K - n_gt

    eq = (key == t).astype(jnp.int32)
    chunks = _N // 128
    e3 = eq.reshape(rows * chunks, 128)
    lane_incl = _lane_cumsum(e3, 128)
    chunk_tot = lane_incl[:, 127:128].reshape(rows, chunks)
    chunk_incl = _lane_cumsum(chunk_tot, chunks)
    chunk_excl = (chunk_incl - chunk_tot).reshape(rows * chunks, 1)
    prefix_excl = (chunk_excl + lane_incl - e3).reshape(rows, _N)

    keep = gt | ((eq > 0) & (prefix_excl < r))
    o_ref[...] = jnp.maximum(jnp.where(keep, xb, 0.0), 0.0)


@jax.jit
def kernel(x):
    m, n = x.shape
    grid = (m // _ROWS_PER_BLOCK,)
    return pl.pallas_call(
        _topk_body,
        grid=grid,
        in_specs=[pl.BlockSpec((_ROWS_PER_BLOCK, n), lambda i: (i, 0))],
        out_specs=pl.BlockSpec((_ROWS_PER_BLOCK, n), lambda i: (i, 0)),
        out_shape=jax.ShapeDtypeStruct((m, n), x.dtype),
    )(x)
